# Initial kernel scaffold; baseline (speedup 1.0000x reference)
import jax
import jax.numpy as jnp
from jax.experimental import pallas as pl


def kernel(x, W_enc, b_enc, b_dec):
    raise NotImplementedError("write your pallas kernel here")


# speedup vs baseline: 1.2309x; 1.2309x over previous
import functools

import jax
import jax.numpy as jnp
from jax.experimental import pallas as pl
from jax.experimental.pallas import tpu as pltpu

_D_INP = 3584
_D_HIDE = 65536
_TOP_K = 7
_BATCH = 256
_BLK = 512
_NTOP = 8

_BIG_I32 = 2**30


def _extract_top(vals, idx, n):
    out_v, out_i = [], []
    v = vals
    for _ in range(n):
        m = jnp.max(v, axis=1, keepdims=True)
        am = jnp.min(jnp.where(v == m, idx, _BIG_I32), axis=1, keepdims=True)
        out_v.append(m)
        out_i.append(am)
        v = jnp.where((v == m) & (idx == am), -jnp.inf, v)
    return jnp.concatenate(out_v, axis=1), jnp.concatenate(out_i, axis=1)


def _enc_body(nblk, blk, x_ref, w_ref, be_ref, bd_ref, topv_ref, topi_ref,
              tv, ti):
    j = pl.program_id(0)

    @pl.when(j == 0)
    def _init():
        tv[...] = jnp.full(tv.shape, -jnp.inf, dtype=tv.dtype)
        ti[...] = jnp.zeros(ti.shape, dtype=ti.dtype)

    xc = x_ref[...] - bd_ref[...]
    h = jax.lax.dot_general(
        xc, w_ref[...], (((1,), (0,)), ((), ())),
        preferred_element_type=jnp.float32,
        precision=jax.lax.Precision.DEFAULT,
    ) + be_ref[...]

    b = h.shape[0]
    col = jax.lax.broadcasted_iota(jnp.int32, (b, blk), 1) + j * blk
    bv, bi = _extract_top(h, col, _NTOP)

    cand_v = jnp.concatenate([tv[...], bv], axis=1)
    cand_i = jnp.concatenate([ti[...], bi], axis=1)
    nv, ni = _extract_top(cand_v, cand_i, _NTOP)
    tv[...] = nv
    ti[...] = ni

    @pl.when(j == nblk - 1)
    def _emit():
        topv_ref[...] = tv[...]
        topi_ref[...] = ti[...]


def _dec_body(nblk, blk, top_k, topv_ref, topi_ref, w_ref, bd_ref,
              feat_ref, recon_ref, acc):
    j = pl.program_id(0)
    b = feat_ref.shape[0]
    col = jax.lax.broadcasted_iota(jnp.int32, (b, blk), 1) + j * blk

    f = jnp.zeros((b, blk), dtype=jnp.float32)
    for k in range(top_k):
        v = jax.nn.relu(topv_ref[:, k:k + 1])
        i = topi_ref[:, k:k + 1]
        f = jnp.where(col == i, v, f)
    feat_ref[...] = f

    contrib = jax.lax.dot_general(
        f, w_ref[...], (((1,), (1,)), ((), ())),
        preferred_element_type=jnp.float32,
        precision=jax.lax.Precision.DEFAULT,
    )

    @pl.when(j == 0)
    def _init():
        acc[...] = jnp.zeros(acc.shape, dtype=acc.dtype)

    acc[...] += contrib

    @pl.when(j == nblk - 1)
    def _emit():
        recon_ref[...] = acc[...] + bd_ref[...]


def _run(x, W_enc, b_enc, b_dec, blk, top_k, interpret=False):
    batch, d_inp = x.shape
    d_hide = W_enc.shape[1]
    nblk = d_hide // blk
    be2 = b_enc.reshape(1, d_hide)
    bd2 = b_dec.reshape(1, d_inp)

    topv, topi = pl.pallas_call(
        functools.partial(_enc_body, nblk, blk),
        grid=(nblk,),
        in_specs=[
            pl.BlockSpec((batch, d_inp), lambda j: (0, 0)),
            pl.BlockSpec((d_inp, blk), lambda j: (0, j)),
            pl.BlockSpec((1, blk), lambda j: (0, j)),
            pl.BlockSpec((1, d_inp), lambda j: (0, 0)),
        ],
        out_specs=[
            pl.BlockSpec((batch, _NTOP), lambda j: (0, 0)),
            pl.BlockSpec((batch, _NTOP), lambda j: (0, 0)),
        ],
        out_shape=[
            jax.ShapeDtypeStruct((batch, _NTOP), jnp.float32),
            jax.ShapeDtypeStruct((batch, _NTOP), jnp.int32),
        ],
        scratch_shapes=[
            pltpu.VMEM((batch, _NTOP), jnp.float32),
            pltpu.VMEM((batch, _NTOP), jnp.int32),
        ],
        interpret=interpret,
    )(x, W_enc, be2, bd2)

    feat, recon = pl.pallas_call(
        functools.partial(_dec_body, nblk, blk, top_k),
        grid=(nblk,),
        in_specs=[
            pl.BlockSpec((batch, _NTOP), lambda j: (0, 0)),
            pl.BlockSpec((batch, _NTOP), lambda j: (0, 0)),
            pl.BlockSpec((d_inp, blk), lambda j: (0, j)),
            pl.BlockSpec((1, d_inp), lambda j: (0, 0)),
        ],
        out_specs=[
            pl.BlockSpec((batch, blk), lambda j: (0, j)),
            pl.BlockSpec((batch, d_inp), lambda j: (0, 0)),
        ],
        out_shape=[
            jax.ShapeDtypeStruct((batch, d_hide), jnp.float32),
            jax.ShapeDtypeStruct((batch, d_inp), jnp.float32),
        ],
        scratch_shapes=[
            pltpu.VMEM((batch, d_inp), jnp.float32),
        ],
        interpret=interpret,
    )(topv, topi, W_enc, bd2)

    return recon, feat


def kernel(x, W_enc, b_enc, b_dec):
    return _run(x, W_enc, b_enc, b_dec, _BLK, _TOP_K)


# speedup vs baseline: 1.7105x; 1.3897x over previous
import functools

import jax
import jax.numpy as jnp
from jax.experimental import pallas as pl
from jax.experimental.pallas import tpu as pltpu

_D_INP = 3584
_D_HIDE = 65536
_TOP_K = 7
_BATCH = 256
_BLK = 512
_NTOP = 7

_BIG_I32 = 2**30


def _enc_body(nblk, blk, x_ref, w_ref, be_ref, bd_ref, topv_ref, topi_ref,
              tv, ti, hb):
    j = pl.program_id(0)

    @pl.when(j == 0)
    def _init():
        tv[...] = jnp.full(tv.shape, -jnp.inf, dtype=tv.dtype)
        ti[...] = jnp.zeros(ti.shape, dtype=ti.dtype)

    xc = x_ref[...] - bd_ref[...]
    h = jax.lax.dot_general(
        xc, w_ref[...], (((1,), (0,)), ((), ())),
        preferred_element_type=jnp.float32,
        precision=jax.lax.Precision.DEFAULT,
    ) + be_ref[...]
    hb[...] = h

    b = h.shape[0]
    col = jax.lax.broadcasted_iota(jnp.int32, (b, blk), 1) + j * blk
    lane = jax.lax.broadcasted_iota(jnp.int32, (b, _NTOP), 1)

    def _maxarg():
        hv = hb[...]
        m = jnp.max(hv, axis=1, keepdims=True)
        am = jnp.min(jnp.where(hv == m, col, _BIG_I32), axis=1, keepdims=True)
        return m, am

    m0, am0 = _maxarg()
    go0 = jnp.any(m0 > tv[:, _NTOP - 1:_NTOP])

    def _round(carry):
        m, am, _ = carry
        tvv = tv[...]
        tii = ti[...]
        pos = jnp.sum((tvv >= m).astype(jnp.int32), axis=1, keepdims=True)
        sh_v = jnp.concatenate([tvv[:, :1], tvv[:, :_NTOP - 1]], axis=1)
        sh_i = jnp.concatenate([tii[:, :1], tii[:, :_NTOP - 1]], axis=1)
        nv = jnp.where(lane < pos, tvv, jnp.where(lane == pos, m, sh_v))
        ni = jnp.where(lane < pos, tii, jnp.where(lane == pos, am, sh_i))
        tv[...] = nv
        ti[...] = ni
        hb[...] = jnp.where(col == am, -jnp.inf, hb[...])
        m2, am2 = _maxarg()
        go2 = jnp.any(m2 > nv[:, _NTOP - 1:_NTOP])
        return m2, am2, go2

    jax.lax.while_loop(lambda c: c[2], _round, (m0, am0, go0))

    @pl.when(j == nblk - 1)
    def _emit():
        topv_ref[...] = tv[...]
        topi_ref[...] = ti[...]


def _dec_body(nblk, blk, top_k, topv_ref, topi_ref, w_ref, bd_ref,
              feat_ref, recon_ref, acc):
    j = pl.program_id(0)
    b = feat_ref.shape[0]
    col = jax.lax.broadcasted_iota(jnp.int32, (b, blk), 1) + j * blk

    f = jnp.zeros((b, blk), dtype=jnp.float32)
    for k in range(top_k):
        v = jax.nn.relu(topv_ref[:, k:k + 1])
        i = topi_ref[:, k:k + 1]
        f = jnp.where(col == i, v, f)
    feat_ref[...] = f

    contrib = jax.lax.dot_general(
        f, w_ref[...], (((1,), (1,)), ((), ())),
        preferred_element_type=jnp.float32,
        precision=jax.lax.Precision.DEFAULT,
    )

    @pl.when(j == 0)
    def _init():
        acc[...] = jnp.zeros(acc.shape, dtype=acc.dtype)

    acc[...] += contrib

    @pl.when(j == nblk - 1)
    def _emit():
        recon_ref[...] = acc[...] + bd_ref[...]


def _run(x, W_enc, b_enc, b_dec, blk, top_k, interpret=False):
    batch, d_inp = x.shape
    d_hide = W_enc.shape[1]
    nblk = d_hide // blk
    be2 = b_enc.reshape(1, d_hide)
    bd2 = b_dec.reshape(1, d_inp)

    topv, topi = pl.pallas_call(
        functools.partial(_enc_body, nblk, blk),
        grid=(nblk,),
        in_specs=[
            pl.BlockSpec((batch, d_inp), lambda j: (0, 0)),
            pl.BlockSpec((d_inp, blk), lambda j: (0, j)),
            pl.BlockSpec((1, blk), lambda j: (0, j)),
            pl.BlockSpec((1, d_inp), lambda j: (0, 0)),
        ],
        out_specs=[
            pl.BlockSpec((batch, _NTOP), lambda j: (0, 0)),
            pl.BlockSpec((batch, _NTOP), lambda j: (0, 0)),
        ],
        out_shape=[
            jax.ShapeDtypeStruct((batch, _NTOP), jnp.float32),
            jax.ShapeDtypeStruct((batch, _NTOP), jnp.int32),
        ],
        scratch_shapes=[
            pltpu.VMEM((batch, _NTOP), jnp.float32),
            pltpu.VMEM((batch, _NTOP), jnp.int32),
            pltpu.VMEM((batch, blk), jnp.float32),
        ],
        interpret=interpret,
    )(x, W_enc, be2, bd2)

    feat, recon = pl.pallas_call(
        functools.partial(_dec_body, nblk, blk, top_k),
        grid=(nblk,),
        in_specs=[
            pl.BlockSpec((batch, _NTOP), lambda j: (0, 0)),
            pl.BlockSpec((batch, _NTOP), lambda j: (0, 0)),
            pl.BlockSpec((d_inp, blk), lambda j: (0, j)),
            pl.BlockSpec((1, d_inp), lambda j: (0, 0)),
        ],
        out_specs=[
            pl.BlockSpec((batch, blk), lambda j: (0, j)),
            pl.BlockSpec((batch, d_inp), lambda j: (0, 0)),
        ],
        out_shape=[
            jax.ShapeDtypeStruct((batch, d_hide), jnp.float32),
            jax.ShapeDtypeStruct((batch, d_inp), jnp.float32),
        ],
        scratch_shapes=[
            pltpu.VMEM((batch, d_inp), jnp.float32),
        ],
        interpret=interpret,
    )(topv, topi, W_enc, bd2)

    return recon, feat


def kernel(x, W_enc, b_enc, b_dec):
    return _run(x, W_enc, b_enc, b_dec, _BLK, _TOP_K)
